# Initial kernel scaffold; baseline (speedup 1.0000x reference)
import jax
import jax.numpy as jnp
from jax.experimental import pallas as pl


def kernel(cosine, norms, label):
    raise NotImplementedError("write your pallas kernel here")


# speedup vs baseline: 12.1938x; 12.1938x over previous
import functools
import math

import jax
import jax.numpy as jnp
from jax.experimental import pallas as pl
from jax.experimental.pallas import tpu as pltpu

M = 0.4
H = 0.333
S = 10.0
EPS = 0.001


def _adaface_kernel(label_ref, norms_ref, cos_ref, loss_ref,
                    sumexp_ref, labval_ref, *, block_c, n_cols, n_rows):
    j = pl.program_id(0)
    num_j = pl.num_programs(0)

    c = jnp.clip(cos_ref[...], -1.0 + EPS, 1.0 - EPS)
    cols = j * block_c + jax.lax.broadcasted_iota(jnp.int32, c.shape, 1)
    valid = cols < n_cols
    e = jnp.where(valid, jnp.exp(c * S - S), 0.0)
    part = jnp.sum(e, axis=1, keepdims=True)

    mask = cols == label_ref[...]
    lv = jnp.sum(jnp.where(mask, c, 0.0), axis=1, keepdims=True)

    @pl.when(j == 0)
    def _init():
        sumexp_ref[...] = part
        labval_ref[...] = lv

    @pl.when(j != 0)
    def _acc():
        sumexp_ref[...] += part
        labval_ref[...] += lv

    @pl.when(j == num_j - 1)
    def _epilogue():
        n = jnp.clip(norms_ref[...], 0.001, 100.0)
        mean = jnp.mean(n)
        var = jnp.sum((n - mean) ** 2) / (n_rows - 1)
        std = jnp.sqrt(var)
        ms = jnp.clip((n - mean) / (std + EPS) * H, -1.0, 1.0)
        g_ang = -M * ms
        g_add = M + M * ms

        c_lab = labval_ref[...]
        g = g_ang
        z_mid = c_lab * jnp.cos(g) - jnp.sqrt(1.0 - c_lab * c_lab) * jnp.sin(g)
        lo = (g < EPS) & (c_lab > jnp.cos(EPS - g))
        hi = (g > -EPS) & (c_lab < -jnp.cos(EPS + g))
        z_clipped = jnp.where(lo, math.cos(EPS),
                              jnp.where(hi, math.cos(math.pi - EPS), z_mid))
        z_new = z_clipped - g_add
        total = sumexp_ref[...] - jnp.exp(S * c_lab - S) + jnp.exp(S * z_new - S)
        loss_i = jnp.log(total) + S - S * z_new
        loss_ref[...] = jnp.mean(loss_i, axis=(0, 1), keepdims=True)


@jax.jit
def kernel(cosine, norms, label):
    n_rows, n_cols = cosine.shape
    block_c = 2048
    num_j = pl.cdiv(n_cols, block_c)
    label2d = label.astype(jnp.int32).reshape(n_rows, 1)

    loss = pl.pallas_call(
        functools.partial(_adaface_kernel, block_c=block_c,
                          n_cols=n_cols, n_rows=n_rows),
        grid=(num_j,),
        in_specs=[
            pl.BlockSpec((n_rows, 1), lambda j: (0, 0)),
            pl.BlockSpec((n_rows, 1), lambda j: (0, 0)),
            pl.BlockSpec((n_rows, block_c), lambda j: (0, j)),
        ],
        out_specs=pl.BlockSpec((1, 1), lambda j: (0, 0)),
        out_shape=jax.ShapeDtypeStruct((1, 1), jnp.float32),
        scratch_shapes=[
            pltpu.VMEM((n_rows, 1), jnp.float32),
            pltpu.VMEM((n_rows, 1), jnp.float32),
        ],
    )(label2d, norms, cosine)
    return loss[0, 0]


# speedup vs baseline: 12.2520x; 1.0048x over previous
import functools
import math

import jax
import jax.numpy as jnp
from jax.experimental import pallas as pl
from jax.experimental.pallas import tpu as pltpu

M = 0.4
H = 0.333
S = 10.0
EPS = 0.001
LANES = 128


def _stream_kernel(label_ref, cos_ref, acc_ref, lab_ref, *, block_c, n_cols, n_rows):
    j = pl.program_id(0)
    base = j * block_c
    iota = jax.lax.broadcasted_iota(jnp.int32, (n_rows, LANES), 1)
    rel = label_ref[...] - base
    lim = n_cols - base

    e_part = None
    l_part = None
    for k in range(block_c // LANES):
        c = jnp.clip(cos_ref[:, k * LANES:(k + 1) * LANES], -1.0 + EPS, 1.0 - EPS)
        e_k = jnp.where(iota < lim - k * LANES, jnp.exp(c * S - S), 0.0)
        lv_k = jnp.where(iota == rel - k * LANES, c, 0.0)
        e_part = e_k if e_part is None else e_part + e_k
        l_part = lv_k if l_part is None else l_part + lv_k

    @pl.when(j == 0)
    def _init():
        acc_ref[...] = e_part
        lab_ref[...] = l_part

    @pl.when(j != 0)
    def _acc():
        acc_ref[...] += e_part
        lab_ref[...] += l_part


def _combine_kernel(norms_ref, acc_ref, lab_ref, loss_ref, *, n_rows):
    n = jnp.clip(norms_ref[...], 0.001, 100.0)
    mean = jnp.mean(n)
    var = jnp.sum((n - mean) ** 2) / (n_rows - 1)
    std = jnp.sqrt(var)
    ms = jnp.clip((n - mean) / (std + EPS) * H, -1.0, 1.0)
    g = -M * ms
    g_add = M + M * ms

    sumexp = jnp.sum(acc_ref[...], axis=1, keepdims=True)
    c_lab = jnp.sum(lab_ref[...], axis=1, keepdims=True)

    z_mid = c_lab * jnp.cos(g) - jnp.sqrt(1.0 - c_lab * c_lab) * jnp.sin(g)
    lo = (g < EPS) & (c_lab > jnp.cos(EPS - g))
    hi = (g > -EPS) & (c_lab < -jnp.cos(EPS + g))
    z_clipped = jnp.where(lo, math.cos(EPS),
                          jnp.where(hi, math.cos(math.pi - EPS), z_mid))
    z_new = z_clipped - g_add
    total = sumexp - jnp.exp(S * c_lab - S) + jnp.exp(S * z_new - S)
    loss_i = jnp.log(total) + S - S * z_new
    loss_ref[...] = jnp.mean(loss_i, axis=(0, 1), keepdims=True)


@jax.jit
def kernel(cosine, norms, label):
    n_rows, n_cols = cosine.shape
    block_c = 2048
    num_j = pl.cdiv(n_cols, block_c)
    label2d = label.astype(jnp.int32).reshape(n_rows, 1)

    acc, lab = pl.pallas_call(
        functools.partial(_stream_kernel, block_c=block_c, n_cols=n_cols,
                          n_rows=n_rows),
        grid=(num_j,),
        in_specs=[
            pl.BlockSpec((n_rows, 1), lambda j: (0, 0)),
            pl.BlockSpec((n_rows, block_c), lambda j: (0, j)),
        ],
        out_specs=[
            pl.BlockSpec((n_rows, LANES), lambda j: (0, 0)),
            pl.BlockSpec((n_rows, LANES), lambda j: (0, 0)),
        ],
        out_shape=[
            jax.ShapeDtypeStruct((n_rows, LANES), jnp.float32),
            jax.ShapeDtypeStruct((n_rows, LANES), jnp.float32),
        ],
    )(label2d, cosine)

    loss = pl.pallas_call(
        functools.partial(_combine_kernel, n_rows=n_rows),
        out_shape=jax.ShapeDtypeStruct((1, 1), jnp.float32),
    )(norms, acc, lab)
    return loss[0, 0]


# speedup vs baseline: 12.4581x; 1.0168x over previous
import functools
import math

import jax
import jax.numpy as jnp
from jax.experimental import pallas as pl
from jax.experimental.pallas import tpu as pltpu

M = 0.4
H = 0.333
S = 10.0
EPS = 0.001
LANES = 128
UNROLL = 4


def _stream_kernel(label_ref, cos_ref, acc_ref, lab_ref, *, n_cols, block_r):
    iota = jax.lax.broadcasted_iota(jnp.int32, (block_r, LANES), 1)
    lab = label_ref[...]
    labb = jnp.broadcast_to(lab, (block_r, LANES))

    n_full = n_cols // LANES
    n_loop = (n_full // UNROLL) * UNROLL

    def chunk(col0, e_acc, l_acc):
        c = jnp.clip(cos_ref[:, pl.ds(col0, LANES)], -1.0 + EPS, 1.0 - EPS)
        e_acc = e_acc + jnp.exp(c * S - S)
        l_acc = l_acc + jnp.where(iota + col0 == labb, c, 0.0)
        return e_acc, l_acc

    def body(i, carry):
        e_acc, l_acc = carry
        base = i * (LANES * UNROLL)
        for u in range(UNROLL):
            e_acc, l_acc = chunk(base + u * LANES, e_acc, l_acc)
        return e_acc, l_acc

    zeros = jnp.zeros((block_r, LANES), jnp.float32)
    e_acc, l_acc = jax.lax.fori_loop(0, n_loop // UNROLL, body, (zeros, zeros))

    for col0 in range(n_loop * LANES, n_full * LANES, LANES):
        e_acc, l_acc = chunk(col0, e_acc, l_acc)

    rem = n_cols - n_full * LANES
    if rem:
        col0 = n_cols - LANES
        keep = iota >= (LANES - rem)
        c = jnp.clip(cos_ref[:, pl.ds(col0, LANES)], -1.0 + EPS, 1.0 - EPS)
        e_acc = e_acc + jnp.where(keep, jnp.exp(c * S - S), 0.0)
        l_acc = l_acc + jnp.where(keep & (iota + col0 == labb), c, 0.0)

    acc_ref[...] = e_acc
    lab_ref[...] = l_acc


def _combine_kernel(norms_ref, acc_ref, lab_ref, loss_ref, *, n_rows):
    n = jnp.clip(norms_ref[...], 0.001, 100.0)
    mean = jnp.mean(n)
    var = jnp.sum((n - mean) ** 2) / (n_rows - 1)
    std = jnp.sqrt(var)
    ms = jnp.clip((n - mean) / (std + EPS) * H, -1.0, 1.0)
    g = -M * ms
    g_add = M + M * ms

    sumexp = jnp.sum(acc_ref[...], axis=1, keepdims=True)
    c_lab = jnp.sum(lab_ref[...], axis=1, keepdims=True)

    z_mid = c_lab * jnp.cos(g) - jnp.sqrt(1.0 - c_lab * c_lab) * jnp.sin(g)
    lo = (g < EPS) & (c_lab > jnp.cos(EPS - g))
    hi = (g > -EPS) & (c_lab < -jnp.cos(EPS + g))
    z_clipped = jnp.where(lo, math.cos(EPS),
                          jnp.where(hi, math.cos(math.pi - EPS), z_mid))
    z_new = z_clipped - g_add
    total = sumexp - jnp.exp(S * c_lab - S) + jnp.exp(S * z_new - S)
    loss_i = jnp.log(total) + S - S * z_new
    loss_ref[...] = jnp.mean(loss_i, axis=(0, 1), keepdims=True)


@jax.jit
def kernel(cosine, norms, label):
    n_rows, n_cols = cosine.shape
    block_r = min(64, n_rows)
    num_i = pl.cdiv(n_rows, block_r)
    label2d = label.astype(jnp.int32).reshape(n_rows, 1)

    acc, lab = pl.pallas_call(
        functools.partial(_stream_kernel, n_cols=n_cols, block_r=block_r),
        grid=(num_i,),
        in_specs=[
            pl.BlockSpec((block_r, 1), lambda i: (i, 0)),
            pl.BlockSpec((block_r, n_cols), lambda i: (i, 0)),
        ],
        out_specs=[
            pl.BlockSpec((block_r, LANES), lambda i: (i, 0)),
            pl.BlockSpec((block_r, LANES), lambda i: (i, 0)),
        ],
        out_shape=[
            jax.ShapeDtypeStruct((n_rows, LANES), jnp.float32),
            jax.ShapeDtypeStruct((n_rows, LANES), jnp.float32),
        ],
    )(label2d, cosine)

    loss = pl.pallas_call(
        functools.partial(_combine_kernel, n_rows=n_rows),
        out_shape=jax.ShapeDtypeStruct((1, 1), jnp.float32),
    )(norms, acc, lab)
    return loss[0, 0]
